# baseline (speedup 1.0000x reference)
import jax
import jax.numpy as jnp
from jax.experimental import pallas as pl


def _copy_body(x_ref, o_ref):
    o_ref[...] = x_ref[...]


def kernel(x):
    flat = x.reshape(5, 4)
    out = pl.pallas_call(
        _copy_body,
        out_shape=jax.ShapeDtypeStruct((5, 4), jnp.float32),
    )(flat)
    return out.reshape(5, 2, 2)
